# Initial kernel scaffold; baseline (speedup 1.0000x reference)
import jax
import jax.numpy as jnp
from jax.experimental import pallas as pl


def kernel(x, edge_index, edge_attr, batch, params):
    raise NotImplementedError("write your pallas kernel here")


# speedup vs baseline: 1.0241x; 1.0241x over previous
import functools

import jax
import jax.numpy as jnp
from jax.experimental import pallas as pl

N = 10000
E = 320000
H = 128
G = 64


def _mlp_body(hpre_ref, w1t_ref, b1_ref, g_ref, be_ref, w2t_ref, b2_ref, out_ref):
    h = hpre_ref[...]
    t = jnp.dot(h, w1t_ref[...], preferred_element_type=jnp.float32) + b1_ref[...]
    mu = jnp.mean(t, axis=0, keepdims=True)
    var = jnp.mean((t - mu) ** 2, axis=0, keepdims=True)
    tn = (t - mu) * jax.lax.rsqrt(var + 1e-5) * g_ref[...] + be_ref[...]
    tn = jnp.maximum(tn, 0.0)
    o = jnp.dot(tn, w2t_ref[...], preferred_element_type=jnp.float32) + b2_ref[...]
    out_ref[...] = jnp.maximum(o, 0.0)


def _mlp(hpre, p):
    n, c = hpre.shape
    return pl.pallas_call(
        _mlp_body,
        out_shape=jax.ShapeDtypeStruct((n, H), jnp.float32),
    )(hpre, p['W1'].T, p['b1'][None, :], p['g'][None, :], p['be'][None, :],
      p['W2'].T, p['b2'][None, :])


def _final_body(h_ref, batch_ref, fcwt_ref, fcb_ref, outwt_ref, outb_ref, out_ref):
    h = h_ref[...]
    onehot = (batch_ref[...] == jax.lax.broadcasted_iota(jnp.int32, (1, G), 1)
              ).astype(jnp.float32)
    pooled = jax.lax.dot_general(onehot, h, (((0,), (0,)), ((), ())),
                                 preferred_element_type=jnp.float32)
    emb = jnp.maximum(
        jnp.dot(pooled, fcwt_ref[...], preferred_element_type=jnp.float32)
        + fcb_ref[...], 0.0)
    out_ref[...] = (jnp.dot(emb, outwt_ref[...], preferred_element_type=jnp.float32)
                    + outb_ref[...])


def _final(h, batch, params):
    return pl.pallas_call(
        _final_body,
        out_shape=jax.ShapeDtypeStruct((G, 2), jnp.float32),
    )(h, batch[:, None], params['fcW'].T, params['fcb'][None, :],
      params['outW'].T, params['outb'][None, :])


def _messages_xla(h, src, dst, ea, p):
    e = ea @ p['leW'].T + p['leb']
    m = jax.nn.relu(h[src] + e)
    return jnp.zeros_like(h).at[dst].add(m)


def kernel(x, edge_index, edge_attr, batch, params):
    src, dst = edge_index[0], edge_index[1]
    h = x
    for name in ('c1', 'c2', 'c3'):
        p = params[name]
        agg = _messages_xla(h, src, dst, edge_attr, p)
        h = _mlp(h + agg, p)
    return _final(h, batch, params)


# speedup vs baseline: 3.7288x; 3.6411x over previous
import functools

import jax
import jax.numpy as jnp
from jax import lax
from jax.experimental import pallas as pl
from jax.experimental.pallas import tpu as pltpu
from jax.experimental.pallas import tpu_sc as plsc

N = 10000
E = 320000
H = 128
G = 64

NC = 2
NS = 16
CHUNK = 80
NCHUNK = E // CHUNK
CPT = NCHUNK // (NC * NS)
GRP = 25
RPT = N // NS
ZPT = 624
ZTAIL = N - NS * ZPT


def _msg_body(D, h_hbm, sd_hbm, ea_hbm, w_hbm, out_hbm,
              w_v, sd_v, ea_v, rows_v, agg_sh, sem):
    KV = D // 16
    cid = lax.axis_index("c")
    sid = lax.axis_index("s")
    gid = cid * NS + sid

    pltpu.sync_copy(w_hbm, w_v)

    zero = jnp.zeros((16,), jnp.float32)

    def _zrow(j, c):
        for k in range(KV):
            rows_v[j, pl.ds(k * 16, 16)] = zero
        return c

    lax.fori_loop(0, CHUNK, _zrow, 0)
    nfull, rem = ZPT // CHUNK, ZPT % CHUNK
    for q in range(nfull):
        pltpu.sync_copy(rows_v,
                        agg_sh.at[pl.ds(sid * ZPT + q * CHUNK, CHUNK)])
    if rem:
        pltpu.sync_copy(rows_v.at[pl.ds(0, rem)],
                        agg_sh.at[pl.ds(sid * ZPT + nfull * CHUNK, rem)])

    @pl.when(sid == 0)
    def _zero_tail():
        pltpu.sync_copy(rows_v.at[pl.ds(0, ZTAIL)],
                        agg_sh.at[pl.ds(NS * ZPT, ZTAIL)])

    plsc.subcore_barrier()

    ws = [[w_v[r, pl.ds(k * 16, 16)] for k in range(KV)] for r in range(4)]

    def _chunk_grp(gi, c):
        base = gid * CPT + gi * GRP
        pltpu.sync_copy(sd_hbm.at[pl.ds(base, GRP)], sd_v)
        pltpu.sync_copy(ea_hbm.at[pl.ds(base, GRP)], ea_v)

        def _chunk(t, c1):
            pltpu.async_copy(h_hbm.at[sd_v.at[t, 0]], rows_v, sem).wait()

            def _group(gq, c2):
                j0 = gq * 16
                a0v = ea_v[t, 0, pl.ds(j0, 16)]
                a1v = ea_v[t, 1, pl.ds(j0, 16)]
                a2v = ea_v[t, 2, pl.ds(j0, 16)]
                for i in range(16):
                    a0, a1, a2 = a0v[i], a1v[i], a2v[i]
                    for k in range(KV):
                        sl = pl.ds(k * 16, 16)
                        v = rows_v[j0 + i, sl]
                        e = (ws[3][k] + a0 * ws[0][k] + a1 * ws[1][k]
                             + a2 * ws[2][k])
                        rows_v[j0 + i, sl] = jnp.maximum(v + e, 0.0)
                return c2

            lax.fori_loop(0, CHUNK // 16, _group, 0)
            pltpu.sync_copy(rows_v, agg_sh.at[sd_v.at[t, 1]], add=True)
            return c1

        lax.fori_loop(0, GRP, _chunk, 0)
        return c

    lax.fori_loop(0, CPT // GRP, _chunk_grp, 0)
    plsc.subcore_barrier()

    pltpu.sync_copy(agg_sh.at[pl.ds(sid * ZPT, ZPT)],
                    out_hbm.at[cid, pl.ds(sid * ZPT, ZPT)])

    @pl.when(sid == 0)
    def _out_tail():
        pltpu.sync_copy(agg_sh.at[pl.ds(NS * ZPT, ZTAIL)],
                        out_hbm.at[cid, pl.ds(NS * ZPT, ZTAIL)])


def _sc_messages(D):
    mesh = plsc.VectorSubcoreMesh(core_axis_name="c", subcore_axis_name="s")
    return pl.kernel(
        functools.partial(_msg_body, D),
        out_type=jax.ShapeDtypeStruct((NC, N, D), jnp.float32),
        mesh=mesh,
        scratch_types=[
            pltpu.VMEM((4, D), jnp.float32),
            pltpu.VMEM((GRP, 2, CHUNK), jnp.int32),
            pltpu.VMEM((GRP, 3, CHUNK), jnp.float32),
            pltpu.VMEM((CHUNK, D), jnp.float32),
            pltpu.VMEM_SHARED((N, D), jnp.float32),
            pltpu.SemaphoreType.DMA,
        ],
    )


def _mlp_body(base_ref, agg_ref, w1t_ref, b1_ref, g_ref, be_ref, w2t_ref,
              b2_ref, out_ref):
    h = base_ref[...] + agg_ref[0] + agg_ref[1]
    t = jnp.dot(h, w1t_ref[...], preferred_element_type=jnp.float32) + b1_ref[...]
    mu = jnp.mean(t, axis=0, keepdims=True)
    var = jnp.mean((t - mu) ** 2, axis=0, keepdims=True)
    tn = (t - mu) * jax.lax.rsqrt(var + 1e-5) * g_ref[...] + be_ref[...]
    tn = jnp.maximum(tn, 0.0)
    o = jnp.dot(tn, w2t_ref[...], preferred_element_type=jnp.float32) + b2_ref[...]
    out_ref[...] = jnp.maximum(o, 0.0)


def _mlp(base, agg, w1t, b1, g, be, w2t, b2):
    return pl.pallas_call(
        _mlp_body,
        out_shape=jax.ShapeDtypeStruct((N, H), jnp.float32),
    )(base, agg, w1t, b1[None, :], g[None, :], be[None, :], w2t, b2[None, :])


def _final_body(h_ref, batch_ref, fcwt_ref, fcb_ref, outwt_ref, outb_ref,
                out_ref):
    h = h_ref[...]
    onehot = (batch_ref[...] == jax.lax.broadcasted_iota(jnp.int32, (1, G), 1)
              ).astype(jnp.float32)
    pooled = jax.lax.dot_general(onehot, h, (((0,), (0,)), ((), ())),
                                 preferred_element_type=jnp.float32)
    emb = jnp.maximum(
        jnp.dot(pooled, fcwt_ref[...], preferred_element_type=jnp.float32)
        + fcb_ref[...], 0.0)
    out_ref[...] = (jnp.dot(emb, outwt_ref[...], preferred_element_type=jnp.float32)
                    + outb_ref[...])


def _final(h, batch, params):
    return pl.pallas_call(
        _final_body,
        out_shape=jax.ShapeDtypeStruct((G, 2), jnp.float32),
    )(h, batch[:, None], params['fcW'].T, params['fcb'][None, :],
      params['outW'].T, params['outb'][None, :])


def _edge_weight_mat(p, D):
    wt = p['leW'].T
    in_c = wt.shape[1]
    w = jnp.concatenate([wt, p['leb'][None, :]], axis=0)
    return jnp.pad(w, ((0, 0), (0, D - in_c)))


def kernel(x, edge_index, edge_attr, batch, params):
    sd = edge_index.reshape(2, NCHUNK, CHUNK).transpose(1, 0, 2)
    ea = edge_attr.T.reshape(3, NCHUNK, CHUNK).transpose(1, 0, 2)

    msg128 = _sc_messages(H)

    x128 = jnp.pad(x, ((0, 0), (0, H - x.shape[1])))
    stack = []
    for name in ('c1', 'c2', 'c3'):
        p = params[name]
        w1t = p['W1'].T
        w1t = jnp.pad(w1t, ((0, H - w1t.shape[0]), (0, 0)))
        stack.append((_edge_weight_mat(p, H), w1t, p['b1'], p['g'],
                      p['be'], p['W2'].T, p['b2']))
    layer_ws = tuple(jnp.stack(z) for z in zip(*stack))

    def _layer(h, wl):
        wedge, w1t, b1, g, be, w2t, b2 = wl
        agg = msg128(h, sd, ea, wedge)
        return _mlp(h, agg, w1t, b1, g, be, w2t, b2), None

    h, _ = lax.scan(_layer, x128, layer_ws)
    return _final(h, batch, params)


# speedup vs baseline: 5.2272x; 1.4019x over previous
import functools

import jax
import jax.numpy as jnp
from jax import lax
from jax.experimental import pallas as pl
from jax.experimental.pallas import tpu as pltpu
from jax.experimental.pallas import tpu_sc as plsc

N = 10000
E = 320000
H = 128
G = 64

NC = 2
NS = 16
CHUNK = 80
NCHUNK = E // CHUNK
CPT = NCHUNK // (NC * NS)
GRP = 25
RPT = N // NS
ZPT = 624
ZTAIL = N - NS * ZPT


def _msg_body(D, h_hbm, sd_hbm, ea_hbm, w_hbm, out_hbm,
              w_v, sd_v, ea_v, rows_v, rows2_v, agg_sh,
              gsem0, gsem1, ssem0, ssem1):
    KV = D // 16
    cid = lax.axis_index("c")
    sid = lax.axis_index("s")
    gid = cid * NS + sid

    pltpu.sync_copy(w_hbm, w_v)

    zero = jnp.zeros((16,), jnp.float32)

    def _zrow(j, c):
        for k in range(KV):
            rows_v[j, pl.ds(k * 16, 16)] = zero
        return c

    lax.fori_loop(0, CHUNK, _zrow, 0)
    nfull, rem = ZPT // CHUNK, ZPT % CHUNK
    for q in range(nfull):
        pltpu.sync_copy(rows_v,
                        agg_sh.at[pl.ds(sid * ZPT + q * CHUNK, CHUNK)])
    if rem:
        pltpu.sync_copy(rows_v.at[pl.ds(0, rem)],
                        agg_sh.at[pl.ds(sid * ZPT + nfull * CHUNK, rem)])

    @pl.when(sid == 0)
    def _zero_tail():
        pltpu.sync_copy(rows_v.at[pl.ds(0, ZTAIL)],
                        agg_sh.at[pl.ds(NS * ZPT, ZTAIL)])

    plsc.subcore_barrier()

    ws = [[w_v[r, pl.ds(k * 16, 16)] for k in range(KV)] for r in range(4)]

    bufs = (rows_v, rows2_v)
    gsems = (gsem0, gsem1)
    ssems = (ssem0, ssem1)

    def _gather(t, b):
        return pltpu.make_async_copy(h_hbm.at[sd_v.at[t, 0]], bufs[b],
                                     gsems[b])

    def _scatter(t, b):
        return pltpu.make_async_copy(bufs[b], agg_sh.at[sd_v.at[t, 1]],
                                     ssems[b])

    def _compute(t, b):
        buf = bufs[b]

        def _group(gq, c2):
            j0 = gq * 16
            a0v = ea_v[t, 0, pl.ds(j0, 16)]
            a1v = ea_v[t, 1, pl.ds(j0, 16)]
            a2v = ea_v[t, 2, pl.ds(j0, 16)]
            for i in range(16):
                a0, a1, a2 = a0v[i], a1v[i], a2v[i]
                for k in range(KV):
                    sl = pl.ds(k * 16, 16)
                    v = buf[j0 + i, sl]
                    e = (ws[3][k] + a0 * ws[0][k] + a1 * ws[1][k]
                         + a2 * ws[2][k])
                    buf[j0 + i, sl] = jnp.maximum(v + e, 0.0)
            return c2

        lax.fori_loop(0, CHUNK // 16, _group, 0)

    def _chunk_grp(gi, c):
        base = gid * CPT + gi * GRP
        pltpu.sync_copy(sd_hbm.at[pl.ds(base, GRP)], sd_v)
        pltpu.sync_copy(ea_hbm.at[pl.ds(base, GRP)], ea_v)

        _gather(0, 0).start()

        def _pair(p, c1):
            t0 = 2 * p

            @pl.when(p > 0)
            def _w():
                _scatter(t0 - 1, 1).wait()

            _gather(t0 + 1, 1).start()
            _gather(t0, 0).wait()
            _compute(t0, 0)
            _scatter(t0, 0).start(add=True)

            _scatter(t0, 0).wait()
            _gather(t0 + 2, 0).start()
            _gather(t0 + 1, 1).wait()
            _compute(t0 + 1, 1)
            _scatter(t0 + 1, 1).start(add=True)
            return c1

        lax.fori_loop(0, (GRP - 1) // 2, _pair, 0)

        _scatter(GRP - 2, 1).wait()
        _gather(GRP - 1, 0).wait()
        _compute(GRP - 1, 0)
        _scatter(GRP - 1, 0).start(add=True)
        _scatter(GRP - 1, 0).wait()
        return c

    lax.fori_loop(0, CPT // GRP, _chunk_grp, 0)
    plsc.subcore_barrier()

    pltpu.sync_copy(agg_sh.at[pl.ds(sid * ZPT, ZPT)],
                    out_hbm.at[cid, pl.ds(sid * ZPT, ZPT)])

    @pl.when(sid == 0)
    def _out_tail():
        pltpu.sync_copy(agg_sh.at[pl.ds(NS * ZPT, ZTAIL)],
                        out_hbm.at[cid, pl.ds(NS * ZPT, ZTAIL)])


def _sc_messages(D):
    mesh = plsc.VectorSubcoreMesh(core_axis_name="c", subcore_axis_name="s")
    return pl.kernel(
        functools.partial(_msg_body, D),
        out_type=jax.ShapeDtypeStruct((NC, N, D), jnp.float32),
        mesh=mesh,
        scratch_types=[
            pltpu.VMEM((4, D), jnp.float32),
            pltpu.VMEM((GRP, 2, CHUNK), jnp.int32),
            pltpu.VMEM((GRP, 3, CHUNK), jnp.float32),
            pltpu.VMEM((CHUNK, D), jnp.float32),
            pltpu.VMEM((CHUNK, D), jnp.float32),
            pltpu.VMEM_SHARED((N, D), jnp.float32),
            pltpu.SemaphoreType.DMA,
            pltpu.SemaphoreType.DMA,
            pltpu.SemaphoreType.DMA,
            pltpu.SemaphoreType.DMA,
        ],
    )


def _mlp_body(base_ref, agg_ref, w1t_ref, b1_ref, g_ref, be_ref, w2t_ref,
              b2_ref, out_ref):
    h = base_ref[...] + agg_ref[0] + agg_ref[1]
    t = jnp.dot(h, w1t_ref[...], preferred_element_type=jnp.float32) + b1_ref[...]
    mu = jnp.mean(t, axis=0, keepdims=True)
    var = jnp.mean((t - mu) ** 2, axis=0, keepdims=True)
    tn = (t - mu) * jax.lax.rsqrt(var + 1e-5) * g_ref[...] + be_ref[...]
    tn = jnp.maximum(tn, 0.0)
    o = jnp.dot(tn, w2t_ref[...], preferred_element_type=jnp.float32) + b2_ref[...]
    out_ref[...] = jnp.maximum(o, 0.0)


def _mlp(base, agg, w1t, b1, g, be, w2t, b2):
    return pl.pallas_call(
        _mlp_body,
        out_shape=jax.ShapeDtypeStruct((N, H), jnp.float32),
    )(base, agg, w1t, b1[None, :], g[None, :], be[None, :], w2t, b2[None, :])


def _final_body(h_ref, batch_ref, fcwt_ref, fcb_ref, outwt_ref, outb_ref,
                out_ref):
    h = h_ref[...]
    onehot = (batch_ref[...] == jax.lax.broadcasted_iota(jnp.int32, (1, G), 1)
              ).astype(jnp.float32)
    pooled = jax.lax.dot_general(onehot, h, (((0,), (0,)), ((), ())),
                                 preferred_element_type=jnp.float32)
    emb = jnp.maximum(
        jnp.dot(pooled, fcwt_ref[...], preferred_element_type=jnp.float32)
        + fcb_ref[...], 0.0)
    out_ref[...] = (jnp.dot(emb, outwt_ref[...], preferred_element_type=jnp.float32)
                    + outb_ref[...])


def _final(h, batch, params):
    return pl.pallas_call(
        _final_body,
        out_shape=jax.ShapeDtypeStruct((G, 2), jnp.float32),
    )(h, batch[:, None], params['fcW'].T, params['fcb'][None, :],
      params['outW'].T, params['outb'][None, :])


def _edge_weight_mat(p, D):
    wt = p['leW'].T
    in_c = wt.shape[1]
    w = jnp.concatenate([wt, p['leb'][None, :]], axis=0)
    return jnp.pad(w, ((0, 0), (0, D - in_c)))


def kernel(x, edge_index, edge_attr, batch, params):
    sd = edge_index.reshape(2, NCHUNK, CHUNK).transpose(1, 0, 2)
    ea = edge_attr.T.reshape(3, NCHUNK, CHUNK).transpose(1, 0, 2)

    msg128 = _sc_messages(H)

    x128 = jnp.pad(x, ((0, 0), (0, H - x.shape[1])))
    stack = []
    for name in ('c1', 'c2', 'c3'):
        p = params[name]
        w1t = p['W1'].T
        w1t = jnp.pad(w1t, ((0, H - w1t.shape[0]), (0, 0)))
        stack.append((_edge_weight_mat(p, H), w1t, p['b1'], p['g'],
                      p['be'], p['W2'].T, p['b2']))
    layer_ws = tuple(jnp.stack(z) for z in zip(*stack))

    def _layer(h, wl):
        wedge, w1t, b1, g, be, w2t, b2 = wl
        agg = msg128(h, sd, ea, wedge)
        return _mlp(h, agg, w1t, b1, g, be, w2t, b2), None

    h, _ = lax.scan(_layer, x128, layer_ws)
    return _final(h, batch, params)
